# baseline (speedup 1.0000x reference)
import functools

import jax
import jax.numpy as jnp
from jax.experimental import pallas as pl
from jax.experimental.pallas import tpu as pltpu

BH = 16
SLOTS = 8


def _relu_pipe(x_hbm, o_hbm, in_buf, out_buf, in_sem, out_sem, *, nb, hb):
    k = pl.program_id(0)

    def in_copy(block, slot):
        bi = block // hb
        j = block % hb
        return pltpu.make_async_copy(
            x_hbm.at[bi, pl.ds(j * BH, BH)],
            in_buf.at[slot],
            in_sem.at[slot],
        )

    def out_copy(block, slot):
        bi = block // hb
        j = block % hb
        return pltpu.make_async_copy(
            out_buf.at[slot],
            o_hbm.at[bi, pl.ds(j * BH, BH)],
            out_sem.at[slot],
        )

    @pl.when(k == 0)
    def _prologue():
        for s in range(SLOTS):
            in_copy(s, s).start()

    slot = jax.lax.rem(k, SLOTS)
    in_copy(k, slot).wait()

    @pl.when(k >= SLOTS)
    def _reclaim_out():
        out_copy(k - SLOTS, slot).wait()

    out_buf[slot] = jnp.maximum(in_buf[slot], 0.0)
    out_copy(k, slot).start()

    @pl.when(k + SLOTS < nb)
    def _next_in():
        in_copy(k + SLOTS, slot).start()

    @pl.when(k == nb - 1)
    def _epilogue():
        for s in range(SLOTS):
            blk = nb - SLOTS + s
            out_copy(blk, blk % SLOTS).wait()


def kernel(x):
    n, h, w, c = x.shape
    hb = h // BH
    nb = n * hb
    out = pl.pallas_call(
        functools.partial(_relu_pipe, nb=nb, hb=hb),
        grid=(nb,),
        in_specs=[pl.BlockSpec(memory_space=pl.ANY)],
        out_specs=pl.BlockSpec(memory_space=pl.ANY),
        out_shape=jax.ShapeDtypeStruct(x.shape, x.dtype),
        scratch_shapes=[
            pltpu.VMEM((SLOTS, BH, w, c), x.dtype),
            pltpu.VMEM((SLOTS, BH, w, c), x.dtype),
            pltpu.SemaphoreType.DMA((SLOTS,)),
            pltpu.SemaphoreType.DMA((SLOTS,)),
        ],
    )(x)
    return out
